# baseline (speedup 1.0000x reference)
import functools

import jax
import jax.numpy as jnp
from jax import lax
from jax.experimental import pallas as pl
from jax.experimental.pallas import tpu as pltpu
from jax.experimental.pallas import tpu_sc as plsc

_N = 10000
_E = 320000
_H = 128
_DOUT = 64

_NCORES = 2
_NSUB = 16
_NTILES = _NCORES * _NSUB
_EPT = _E // _NTILES
_EC = 80
_NEC = 128
_NPAD = _NEC * _EC - _EPT
_HCH = _NEC // 2
_NZ = 16
_RC = 80
_NRC = _N // _RC
_RCPS = (_NRC + _NSUB - 1) // _NSUB

_BLK = 1000
_NB = _N // _BLK


def _segsum_body(h_hbm, src_hbm, dst_hbm, out_hbm, src_v, dst_v, rows_v,
                 acc, gsa, gsb, ssa, ssb):
    c = lax.axis_index("c")
    s = lax.axis_index("s")
    wid = c * _NSUB + s

    zeros16 = jnp.zeros((16,), jnp.float32)

    def _zrow(i, carry):
        for j in range(_H // 16):
            rows_v[0, i, pl.ds(j * 16, 16)] = zeros16
        return carry

    lax.fori_loop(0, _RC, _zrow, 0)

    def _zchunk(j, carry):
        chunk = j * _NSUB + s

        @pl.when(chunk < _NRC)
        def _():
            pltpu.sync_copy(rows_v.at[0], acc.at[pl.ds(chunk * _RC, _RC)])

        return carry

    lax.fori_loop(0, _RCPS, _zchunk, 0)
    plsc.subcore_barrier()

    def _pair(i, carry):
        j0 = 2 * i
        g0 = pltpu.async_copy(h_hbm.at[src_v.at[j0]], rows_v.at[0], gsa)
        g0.wait()
        s0 = pltpu.async_copy(rows_v.at[0], acc.at[dst_v.at[j0]], ssa,
                              add=True)
        g1 = pltpu.async_copy(h_hbm.at[src_v.at[j0 + 1]], rows_v.at[1], gsb)
        g1.wait()
        s0.wait()
        s1 = pltpu.async_copy(rows_v.at[1], acc.at[dst_v.at[j0 + 1]], ssb,
                              add=True)
        s1.wait()
        return carry

    for half in range(2):
        pltpu.sync_copy(src_hbm.at[wid, pl.ds(half * _HCH, _HCH)], src_v)
        pltpu.sync_copy(dst_hbm.at[wid, pl.ds(half * _HCH, _HCH)], dst_v)
        lax.fori_loop(0, _HCH // 2, _pair, 0)

    plsc.subcore_barrier()

    def _ochunk(j, carry):
        chunk = j * _NSUB + s

        @pl.when(chunk < _NRC)
        def _():
            r0 = chunk * _RC
            pltpu.sync_copy(acc.at[pl.ds(r0, _RC)], rows_v.at[0])
            pltpu.sync_copy(rows_v.at[0], out_hbm.at[pl.ds(c * _N + r0, _RC)])

        return carry

    lax.fori_loop(0, _RCPS, _ochunk, 0)


@functools.cache
def _make_segsum():
    return functools.partial(
        pl.kernel,
        mesh=plsc.VectorSubcoreMesh(core_axis_name="c", subcore_axis_name="s"),
        out_type=jax.ShapeDtypeStruct((2 * _N, _H), jnp.float32),
        scratch_types=[
            pltpu.VMEM((_HCH, _EC), jnp.int32),
            pltpu.VMEM((_HCH, _EC), jnp.int32),
            pltpu.VMEM((2, _EC, _H), jnp.float32),
            pltpu.VMEM_SHARED((_N, _H), jnp.float32),
            pltpu.SemaphoreType.DMA,
            pltpu.SemaphoreType.DMA,
            pltpu.SemaphoreType.DMA,
            pltpu.SemaphoreType.DMA,
        ],
    )(_segsum_body)


def _round_a_body(h_ref, p_ref, wa_ref, ba_ref, wr_ref, br_ref,
                  t_ref, id_ref, st_ref):
    j = pl.program_id(0)
    h = h_ref[...]
    agg = h + p_ref[0] + p_ref[1]
    t = jnp.dot(agg, wa_ref[...], preferred_element_type=jnp.float32) + ba_ref[...]
    t_ref[...] = t
    id_ref[...] = jnp.dot(h, wr_ref[...], preferred_element_type=jnp.float32) + br_ref[...]
    blk = jnp.concatenate(
        [jnp.sum(t, axis=0, keepdims=True),
         jnp.sum(t * t, axis=0, keepdims=True)], axis=0)

    @pl.when(j == 0)
    def _():
        st_ref[...] = blk

    @pl.when(j > 0)
    def _():
        st_ref[...] += blk


def _round_a(h, parts, wa, ba, wr, br):
    return pl.pallas_call(
        _round_a_body,
        grid=(_NB,),
        in_specs=[
            pl.BlockSpec((_BLK, _H), lambda j: (j, 0)),
            pl.BlockSpec((2, _BLK, _H), lambda j: (0, j, 0)),
            pl.BlockSpec((_H, _H), lambda j: (0, 0)),
            pl.BlockSpec((1, _H), lambda j: (0, 0)),
            pl.BlockSpec((_H, _H), lambda j: (0, 0)),
            pl.BlockSpec((1, _H), lambda j: (0, 0)),
        ],
        out_specs=[
            pl.BlockSpec((_BLK, _H), lambda j: (j, 0)),
            pl.BlockSpec((_BLK, _H), lambda j: (j, 0)),
            pl.BlockSpec((2, _H), lambda j: (0, 0)),
        ],
        out_shape=[
            jax.ShapeDtypeStruct((_N, _H), jnp.float32),
            jax.ShapeDtypeStruct((_N, _H), jnp.float32),
            jax.ShapeDtypeStruct((2, _H), jnp.float32),
        ],
    )(h, parts, wa, ba.reshape(1, _H), wr, br.reshape(1, _H))


def _round_b_body(t_ref, id_ref, st_ref, g_ref, be_ref, wb_ref, bb_ref,
                  wc_ref, bc_ref, o_ref):
    st = st_ref[...]
    m = st[0:1] * (1.0 / _N)
    v = st[1:2] * (1.0 / _N) - m * m
    inv = lax.rsqrt(v + 1e-5)
    u = (t_ref[...] - m) * (inv * g_ref[...]) + be_ref[...]
    u = jnp.maximum(u, 0.0)
    u = jnp.maximum(
        jnp.dot(u, wb_ref[...], preferred_element_type=jnp.float32) + bb_ref[...], 0.0)
    hh = jnp.dot(u, wc_ref[...], preferred_element_type=jnp.float32) + bc_ref[...]
    o_ref[...] = jnp.maximum(hh + id_ref[...], 0.0)


def _round_b(t, ident, stats, g, be, wb, bb, wc, bc):
    return pl.pallas_call(
        _round_b_body,
        grid=(_NB,),
        in_specs=[
            pl.BlockSpec((_BLK, _H), lambda j: (j, 0)),
            pl.BlockSpec((_BLK, _H), lambda j: (j, 0)),
            pl.BlockSpec((2, _H), lambda j: (0, 0)),
            pl.BlockSpec((1, _H), lambda j: (0, 0)),
            pl.BlockSpec((1, _H), lambda j: (0, 0)),
            pl.BlockSpec((_H, _H), lambda j: (0, 0)),
            pl.BlockSpec((1, _H), lambda j: (0, 0)),
            pl.BlockSpec((_H, _H), lambda j: (0, 0)),
            pl.BlockSpec((1, _H), lambda j: (0, 0)),
        ],
        out_specs=pl.BlockSpec((_BLK, _H), lambda j: (j, 0)),
        out_shape=jax.ShapeDtypeStruct((_N, _H), jnp.float32),
    )(t, ident, stats, g.reshape(1, _H), be.reshape(1, _H),
      wb, bb.reshape(1, _H), wc, bc.reshape(1, _H))


def _final_body(h_ref, p_ref, w4_ref, b4_ref, o_ref):
    agg = h_ref[...] + p_ref[0] + p_ref[1]
    z = jnp.dot(agg, w4_ref[...], preferred_element_type=jnp.float32) + b4_ref[...]
    mx = jnp.max(z, axis=1, keepdims=True)
    e = jnp.exp(z - mx)
    o_ref[...] = z - mx - jnp.log(jnp.sum(e, axis=1, keepdims=True))


def _final(h, parts, w4, b4):
    return pl.pallas_call(
        _final_body,
        grid=(_NB,),
        in_specs=[
            pl.BlockSpec((_BLK, _H), lambda j: (j, 0)),
            pl.BlockSpec((2, _BLK, _H), lambda j: (0, j, 0)),
            pl.BlockSpec((_H, _DOUT), lambda j: (0, 0)),
            pl.BlockSpec((1, _DOUT), lambda j: (0, 0)),
        ],
        out_specs=pl.BlockSpec((_BLK, _DOUT), lambda j: (j, 0)),
        out_shape=jax.ShapeDtypeStruct((_N, _DOUT), jnp.float32),
    )(h, parts, w4, b4.reshape(1, _DOUT))


def kernel(x, edge_index,
           W1a, b1a, g1, be1, W1b, b1b, W1c, b1c,
           W2a, b2a, g2, be2, W2b, b2b, W2c, b2c,
           W3a, b3a, g3, be3, W3b, b3b, W3c, b3c,
           W4, b4, Wr1, br1, Wr2, br2, Wr3, br3):
    src = jnp.concatenate(
        [edge_index[0].reshape(_NTILES, _EPT),
         jnp.full((_NTILES, _NPAD), _N, jnp.int32)], axis=1).reshape(
             _NTILES, _NEC, _EC)
    pad_dst = (jnp.arange(_NTILES, dtype=jnp.int32)[:, None] * _NPAD
               + jnp.arange(_NPAD, dtype=jnp.int32)[None, :])
    dst = jnp.concatenate(
        [edge_index[1].reshape(_NTILES, _EPT), pad_dst],
        axis=1).reshape(_NTILES, _NEC, _EC)

    segsum = _make_segsum()
    zrows = jnp.zeros((_NZ, _H), jnp.float32)

    def agg_parts(h):
        hz = jnp.concatenate([h, zrows], axis=0)
        return segsum(hz, src, dst).reshape(2, _N, _H)

    h = x
    for (wa, ba, g, be, wb, bb, wc, bc, wr, br) in (
            (W1a, b1a, g1, be1, W1b, b1b, W1c, b1c, Wr1, br1),
            (W2a, b2a, g2, be2, W2b, b2b, W2c, b2c, Wr2, br2),
            (W3a, b3a, g3, be3, W3b, b3b, W3c, b3c, Wr3, br3)):
        parts = agg_parts(h)
        t, ident, stats = _round_a(h, parts, wa, ba, wr, br)
        h = _round_b(t, ident, stats, g, be, wb, bb, wc, bc)
    return _final(h, agg_parts(h), W4, b4)


# speedup vs baseline: 2.1230x; 2.1230x over previous
import functools

import jax
import jax.numpy as jnp
from jax import lax
from jax.experimental import pallas as pl
from jax.experimental.pallas import tpu as pltpu
from jax.experimental.pallas import tpu_sc as plsc

_N = 10000
_E = 320000
_H = 128
_DOUT = 64

_NCORES = 2
_NSUB = 16
_NTILES = _NCORES * _NSUB
_EPT = _E // _NTILES
_EC = 80
_NEC = _EPT // _EC
_RC = 80
_NRC = _N // _RC
_RCPS = (_NRC + _NSUB - 1) // _NSUB

_BLK = 1000
_NB = _N // _BLK


def _segsum_body(h_hbm, src_hbm, dst_hbm, out_hbm, src_v, dst_v, rows_v,
                 acc, sem):
    c = lax.axis_index("c")
    s = lax.axis_index("s")
    wid = c * _NSUB + s

    zeros16 = jnp.zeros((16,), jnp.float32)

    def _zrow(i, carry):
        for j in range(_H // 16):
            rows_v[i, pl.ds(j * 16, 16)] = zeros16
        return carry

    lax.fori_loop(0, _RC, _zrow, 0)

    def _zchunk(j, carry):
        chunk = j * _NSUB + s

        @pl.when(chunk < _NRC)
        def _():
            pltpu.sync_copy(rows_v, acc.at[pl.ds(chunk * _RC, _RC)])

        return carry

    lax.fori_loop(0, _RCPS, _zchunk, 0)

    pltpu.sync_copy(src_hbm.at[wid], src_v)
    pltpu.sync_copy(dst_hbm.at[wid], dst_v)
    plsc.subcore_barrier()

    def _echunk(j, carry):
        pltpu.async_copy(h_hbm.at[src_v.at[j]], rows_v, sem).wait()
        pltpu.sync_copy(rows_v, acc.at[dst_v.at[j]], add=True)
        return carry

    lax.fori_loop(0, _NEC, _echunk, 0)
    plsc.subcore_barrier()

    def _ochunk(j, carry):
        chunk = j * _NSUB + s

        @pl.when(chunk < _NRC)
        def _():
            r0 = chunk * _RC
            pltpu.sync_copy(acc.at[pl.ds(r0, _RC)], rows_v)
            pltpu.sync_copy(rows_v, out_hbm.at[pl.ds(c * _N + r0, _RC)])

        return carry

    lax.fori_loop(0, _RCPS, _ochunk, 0)


@functools.cache
def _make_segsum():
    return functools.partial(
        pl.kernel,
        mesh=plsc.VectorSubcoreMesh(core_axis_name="c", subcore_axis_name="s"),
        out_type=jax.ShapeDtypeStruct((2 * _N, _H), jnp.float32),
        scratch_types=[
            pltpu.VMEM((_NEC, _EC), jnp.int32),
            pltpu.VMEM((_NEC, _EC), jnp.int32),
            pltpu.VMEM((_EC, _H), jnp.float32),
            pltpu.VMEM_SHARED((_N, _H), jnp.float32),
            pltpu.SemaphoreType.DMA,
        ],
    )(_segsum_body)


def _round_a_body(h_ref, p_ref, wa_ref, ba_ref, wr_ref, br_ref,
                  t_ref, id_ref, st_ref):
    j = pl.program_id(0)
    h = h_ref[...]
    agg = h + p_ref[0] + p_ref[1]
    t = jnp.dot(agg, wa_ref[...], preferred_element_type=jnp.float32) + ba_ref[...]
    t_ref[...] = t
    id_ref[...] = jnp.dot(h, wr_ref[...], preferred_element_type=jnp.float32) + br_ref[...]
    blk = jnp.concatenate(
        [jnp.sum(t, axis=0, keepdims=True),
         jnp.sum(t * t, axis=0, keepdims=True)], axis=0)

    @pl.when(j == 0)
    def _():
        st_ref[...] = blk

    @pl.when(j > 0)
    def _():
        st_ref[...] += blk


def _round_a(h, parts, wa, ba, wr, br):
    return pl.pallas_call(
        _round_a_body,
        grid=(_NB,),
        in_specs=[
            pl.BlockSpec((_BLK, _H), lambda j: (j, 0)),
            pl.BlockSpec((2, _BLK, _H), lambda j: (0, j, 0)),
            pl.BlockSpec((_H, _H), lambda j: (0, 0)),
            pl.BlockSpec((1, _H), lambda j: (0, 0)),
            pl.BlockSpec((_H, _H), lambda j: (0, 0)),
            pl.BlockSpec((1, _H), lambda j: (0, 0)),
        ],
        out_specs=[
            pl.BlockSpec((_BLK, _H), lambda j: (j, 0)),
            pl.BlockSpec((_BLK, _H), lambda j: (j, 0)),
            pl.BlockSpec((2, _H), lambda j: (0, 0)),
        ],
        out_shape=[
            jax.ShapeDtypeStruct((_N, _H), jnp.float32),
            jax.ShapeDtypeStruct((_N, _H), jnp.float32),
            jax.ShapeDtypeStruct((2, _H), jnp.float32),
        ],
    )(h, parts, wa, ba.reshape(1, _H), wr, br.reshape(1, _H))


def _round_b_body(t_ref, id_ref, st_ref, g_ref, be_ref, wb_ref, bb_ref,
                  wc_ref, bc_ref, o_ref):
    st = st_ref[...]
    m = st[0:1] * (1.0 / _N)
    v = st[1:2] * (1.0 / _N) - m * m
    inv = lax.rsqrt(v + 1e-5)
    u = (t_ref[...] - m) * (inv * g_ref[...]) + be_ref[...]
    u = jnp.maximum(u, 0.0)
    u = jnp.maximum(
        jnp.dot(u, wb_ref[...], preferred_element_type=jnp.float32) + bb_ref[...], 0.0)
    hh = jnp.dot(u, wc_ref[...], preferred_element_type=jnp.float32) + bc_ref[...]
    o_ref[...] = jnp.maximum(hh + id_ref[...], 0.0)


def _round_b(t, ident, stats, g, be, wb, bb, wc, bc):
    return pl.pallas_call(
        _round_b_body,
        grid=(_NB,),
        in_specs=[
            pl.BlockSpec((_BLK, _H), lambda j: (j, 0)),
            pl.BlockSpec((_BLK, _H), lambda j: (j, 0)),
            pl.BlockSpec((2, _H), lambda j: (0, 0)),
            pl.BlockSpec((1, _H), lambda j: (0, 0)),
            pl.BlockSpec((1, _H), lambda j: (0, 0)),
            pl.BlockSpec((_H, _H), lambda j: (0, 0)),
            pl.BlockSpec((1, _H), lambda j: (0, 0)),
            pl.BlockSpec((_H, _H), lambda j: (0, 0)),
            pl.BlockSpec((1, _H), lambda j: (0, 0)),
        ],
        out_specs=pl.BlockSpec((_BLK, _H), lambda j: (j, 0)),
        out_shape=jax.ShapeDtypeStruct((_N, _H), jnp.float32),
    )(t, ident, stats, g.reshape(1, _H), be.reshape(1, _H),
      wb, bb.reshape(1, _H), wc, bc.reshape(1, _H))


def _final_body(h_ref, p_ref, w4_ref, b4_ref, o_ref):
    agg = h_ref[...] + p_ref[0] + p_ref[1]
    z = jnp.dot(agg, w4_ref[...], preferred_element_type=jnp.float32) + b4_ref[...]
    mx = jnp.max(z, axis=1, keepdims=True)
    e = jnp.exp(z - mx)
    o_ref[...] = z - mx - jnp.log(jnp.sum(e, axis=1, keepdims=True))


def _final(h, parts, w4, b4):
    return pl.pallas_call(
        _final_body,
        grid=(_NB,),
        in_specs=[
            pl.BlockSpec((_BLK, _H), lambda j: (j, 0)),
            pl.BlockSpec((2, _BLK, _H), lambda j: (0, j, 0)),
            pl.BlockSpec((_H, _DOUT), lambda j: (0, 0)),
            pl.BlockSpec((1, _DOUT), lambda j: (0, 0)),
        ],
        out_specs=pl.BlockSpec((_BLK, _DOUT), lambda j: (j, 0)),
        out_shape=jax.ShapeDtypeStruct((_N, _DOUT), jnp.float32),
    )(h, parts, w4, b4.reshape(1, _DOUT))


def kernel(x, edge_index,
           W1a, b1a, g1, be1, W1b, b1b, W1c, b1c,
           W2a, b2a, g2, be2, W2b, b2b, W2c, b2c,
           W3a, b3a, g3, be3, W3b, b3b, W3c, b3c,
           W4, b4, Wr1, br1, Wr2, br2, Wr3, br3):
    src = edge_index[0].reshape(_NTILES, _NEC, _EC)
    dst = edge_index[1].reshape(_NTILES, _NEC, _EC)

    segsum = _make_segsum()

    def agg_parts(h):
        return segsum(h, src, dst).reshape(2, _N, _H)

    h = x
    for (wa, ba, g, be, wb, bb, wc, bc, wr, br) in (
            (W1a, b1a, g1, be1, W1b, b1b, W1c, b1c, Wr1, br1),
            (W2a, b2a, g2, be2, W2b, b2b, W2c, b2c, Wr2, br2),
            (W3a, b3a, g3, be3, W3b, b3b, W3c, b3c, Wr3, br3)):
        parts = agg_parts(h)
        t, ident, stats = _round_a(h, parts, wa, ba, wr, br)
        h = _round_b(t, ident, stats, g, be, wb, bb, wc, bc)
    return _final(h, agg_parts(h), W4, b4)


# speedup vs baseline: 2.2369x; 1.0536x over previous
import functools

import jax
import jax.numpy as jnp
from jax import lax
from jax.experimental import pallas as pl
from jax.experimental.pallas import tpu as pltpu
from jax.experimental.pallas import tpu_sc as plsc

_N = 10000
_E = 320000
_H = 128
_DOUT = 64

_NCORES = 2
_NSUB = 16
_NTILES = _NCORES * _NSUB
_EPT = _E // _NTILES
_EC = 80
_NEC = _EPT // _EC
_RC = 80
_NRC = _N // _RC
_RCPS = (_NRC + _NSUB - 1) // _NSUB

_BLK = 1000
_NB = _N // _BLK


def _make_segsum(w=_H):
    def _segsum_body(h_hbm, src_hbm, dst_hbm, out_hbm, src_v, dst_v, rows_v,
                     acc, sem):
        c = lax.axis_index("c")
        s = lax.axis_index("s")
        wid = c * _NSUB + s

        zeros16 = jnp.zeros((16,), jnp.float32)

        def _zrow(i, carry):
            for j in range(w // 16):
                rows_v[i, pl.ds(j * 16, 16)] = zeros16
            return carry

        lax.fori_loop(0, _RC, _zrow, 0)

        def _zchunk(j, carry):
            chunk = j * _NSUB + s

            @pl.when(chunk < _NRC)
            def _():
                pltpu.sync_copy(rows_v, acc.at[pl.ds(chunk * _RC, _RC)])

            return carry

        lax.fori_loop(0, _RCPS, _zchunk, 0)

        pltpu.sync_copy(src_hbm.at[wid], src_v)
        pltpu.sync_copy(dst_hbm.at[wid], dst_v)
        plsc.subcore_barrier()

        def _echunk(j, carry):
            pltpu.async_copy(h_hbm.at[src_v.at[j]], rows_v, sem).wait()
            pltpu.sync_copy(rows_v, acc.at[dst_v.at[j]], add=True)
            return carry

        lax.fori_loop(0, _NEC, _echunk, 0)
        plsc.subcore_barrier()

        def _ochunk(j, carry):
            chunk = j * _NSUB + s

            @pl.when(chunk < _NRC)
            def _():
                r0 = chunk * _RC
                pltpu.sync_copy(acc.at[pl.ds(r0, _RC)], rows_v)
                pltpu.sync_copy(rows_v, out_hbm.at[pl.ds(c * _N + r0, _RC)])

            return carry

        lax.fori_loop(0, _RCPS, _ochunk, 0)

    return functools.partial(
        pl.kernel,
        mesh=plsc.VectorSubcoreMesh(core_axis_name="c", subcore_axis_name="s"),
        compiler_params=pltpu.CompilerParams(use_tc_tiling_on_sc=False),
        out_type=jax.ShapeDtypeStruct((2 * _N, w), jnp.float32),
        scratch_types=[
            pltpu.VMEM((_NEC, _EC), jnp.int32),
            pltpu.VMEM((_NEC, _EC), jnp.int32),
            pltpu.VMEM((_EC, w), jnp.float32),
            pltpu.VMEM_SHARED((_N, w), jnp.float32),
            pltpu.SemaphoreType.DMA,
        ],
    )(_segsum_body)


_make_segsum = functools.cache(_make_segsum)


def _round_a_body(h_ref, p_ref, wa_ref, ba_ref, wr_ref, br_ref,
                  t_ref, id_ref, st_ref):
    j = pl.program_id(0)
    h = h_ref[...]
    agg = h + p_ref[0] + p_ref[1]
    t = jnp.dot(agg, wa_ref[...], preferred_element_type=jnp.float32) + ba_ref[...]
    t_ref[...] = t
    id_ref[...] = jnp.dot(h, wr_ref[...], preferred_element_type=jnp.float32) + br_ref[...]
    blk = jnp.concatenate(
        [jnp.sum(t, axis=0, keepdims=True),
         jnp.sum(t * t, axis=0, keepdims=True)], axis=0)

    @pl.when(j == 0)
    def _():
        st_ref[...] = blk

    @pl.when(j > 0)
    def _():
        st_ref[...] += blk


def _round_a(h, parts, wa, ba, wr, br):
    return pl.pallas_call(
        _round_a_body,
        grid=(_NB,),
        in_specs=[
            pl.BlockSpec((_BLK, _H), lambda j: (j, 0)),
            pl.BlockSpec((2, _BLK, _H), lambda j: (0, j, 0)),
            pl.BlockSpec((_H, _H), lambda j: (0, 0)),
            pl.BlockSpec((1, _H), lambda j: (0, 0)),
            pl.BlockSpec((_H, _H), lambda j: (0, 0)),
            pl.BlockSpec((1, _H), lambda j: (0, 0)),
        ],
        out_specs=[
            pl.BlockSpec((_BLK, _H), lambda j: (j, 0)),
            pl.BlockSpec((_BLK, _H), lambda j: (j, 0)),
            pl.BlockSpec((2, _H), lambda j: (0, 0)),
        ],
        out_shape=[
            jax.ShapeDtypeStruct((_N, _H), jnp.float32),
            jax.ShapeDtypeStruct((_N, _H), jnp.float32),
            jax.ShapeDtypeStruct((2, _H), jnp.float32),
        ],
    )(h, parts, wa, ba.reshape(1, _H), wr, br.reshape(1, _H))


def _round_b_body(t_ref, id_ref, st_ref, g_ref, be_ref, wb_ref, bb_ref,
                  wc_ref, bc_ref, o_ref):
    st = st_ref[...]
    m = st[0:1] * (1.0 / _N)
    v = st[1:2] * (1.0 / _N) - m * m
    inv = lax.rsqrt(v + 1e-5)
    u = (t_ref[...] - m) * (inv * g_ref[...]) + be_ref[...]
    u = jnp.maximum(u, 0.0)
    u = jnp.maximum(
        jnp.dot(u, wb_ref[...], preferred_element_type=jnp.float32) + bb_ref[...], 0.0)
    hh = jnp.dot(u, wc_ref[...], preferred_element_type=jnp.float32) + bc_ref[...]
    o_ref[...] = jnp.maximum(hh + id_ref[...], 0.0)


def _round_b(t, ident, stats, g, be, wb, bb, wc, bc):
    return pl.pallas_call(
        _round_b_body,
        grid=(_NB,),
        in_specs=[
            pl.BlockSpec((_BLK, _H), lambda j: (j, 0)),
            pl.BlockSpec((_BLK, _H), lambda j: (j, 0)),
            pl.BlockSpec((2, _H), lambda j: (0, 0)),
            pl.BlockSpec((1, _H), lambda j: (0, 0)),
            pl.BlockSpec((1, _H), lambda j: (0, 0)),
            pl.BlockSpec((_H, _H), lambda j: (0, 0)),
            pl.BlockSpec((1, _H), lambda j: (0, 0)),
            pl.BlockSpec((_H, _H), lambda j: (0, 0)),
            pl.BlockSpec((1, _H), lambda j: (0, 0)),
        ],
        out_specs=pl.BlockSpec((_BLK, _H), lambda j: (j, 0)),
        out_shape=jax.ShapeDtypeStruct((_N, _H), jnp.float32),
    )(t, ident, stats, g.reshape(1, _H), be.reshape(1, _H),
      wb, bb.reshape(1, _H), wc, bc.reshape(1, _H))


def _round_b3_body(t_ref, id_ref, st_ref, g_ref, be_ref, wb_ref, bb_ref,
                   wc_ref, bc_ref, w4_ref, o_ref, p_ref):
    st = st_ref[...]
    m = st[0:1] * (1.0 / _N)
    v = st[1:2] * (1.0 / _N) - m * m
    inv = lax.rsqrt(v + 1e-5)
    u = (t_ref[...] - m) * (inv * g_ref[...]) + be_ref[...]
    u = jnp.maximum(u, 0.0)
    u = jnp.maximum(
        jnp.dot(u, wb_ref[...], preferred_element_type=jnp.float32) + bb_ref[...], 0.0)
    hh = jnp.dot(u, wc_ref[...], preferred_element_type=jnp.float32) + bc_ref[...]
    h3 = jnp.maximum(hh + id_ref[...], 0.0)
    o_ref[...] = h3
    p_ref[...] = jnp.dot(h3, w4_ref[...], preferred_element_type=jnp.float32)


def _round_b3(t, ident, stats, g, be, wb, bb, wc, bc, w4):
    return pl.pallas_call(
        _round_b3_body,
        grid=(_NB,),
        in_specs=[
            pl.BlockSpec((_BLK, _H), lambda j: (j, 0)),
            pl.BlockSpec((_BLK, _H), lambda j: (j, 0)),
            pl.BlockSpec((2, _H), lambda j: (0, 0)),
            pl.BlockSpec((1, _H), lambda j: (0, 0)),
            pl.BlockSpec((1, _H), lambda j: (0, 0)),
            pl.BlockSpec((_H, _H), lambda j: (0, 0)),
            pl.BlockSpec((1, _H), lambda j: (0, 0)),
            pl.BlockSpec((_H, _H), lambda j: (0, 0)),
            pl.BlockSpec((1, _H), lambda j: (0, 0)),
            pl.BlockSpec((_H, _DOUT), lambda j: (0, 0)),
        ],
        out_specs=[
            pl.BlockSpec((_BLK, _H), lambda j: (j, 0)),
            pl.BlockSpec((_BLK, _DOUT), lambda j: (j, 0)),
        ],
        out_shape=[
            jax.ShapeDtypeStruct((_N, _H), jnp.float32),
            jax.ShapeDtypeStruct((_N, _DOUT), jnp.float32),
        ],
    )(t, ident, stats, g.reshape(1, _H), be.reshape(1, _H),
      wb, bb.reshape(1, _H), wc, bc.reshape(1, _H), w4)


def _final_body(p_ref, parts_ref, b4_ref, o_ref):
    z = p_ref[...] + parts_ref[0] + parts_ref[1] + b4_ref[...]
    mx = jnp.max(z, axis=1, keepdims=True)
    e = jnp.exp(z - mx)
    o_ref[...] = z - mx - jnp.log(jnp.sum(e, axis=1, keepdims=True))


def _final(p4, parts, b4):
    return pl.pallas_call(
        _final_body,
        grid=(_NB,),
        in_specs=[
            pl.BlockSpec((_BLK, _DOUT), lambda j: (j, 0)),
            pl.BlockSpec((2, _BLK, _DOUT), lambda j: (0, j, 0)),
            pl.BlockSpec((1, _DOUT), lambda j: (0, 0)),
        ],
        out_specs=pl.BlockSpec((_BLK, _DOUT), lambda j: (j, 0)),
        out_shape=jax.ShapeDtypeStruct((_N, _DOUT), jnp.float32),
    )(p4, parts, b4.reshape(1, _DOUT))


def kernel(x, edge_index,
           W1a, b1a, g1, be1, W1b, b1b, W1c, b1c,
           W2a, b2a, g2, be2, W2b, b2b, W2c, b2c,
           W3a, b3a, g3, be3, W3b, b3b, W3c, b3c,
           W4, b4, Wr1, br1, Wr2, br2, Wr3, br3):
    src = edge_index[0].reshape(_NTILES, _NEC, _EC)
    dst = edge_index[1].reshape(_NTILES, _NEC, _EC)

    segsum = _make_segsum()
    segsum64 = _make_segsum(_DOUT)

    def agg_parts(h):
        return segsum(h, src, dst).reshape(2, _N, _H)

    h = x
    p4 = None
    rounds = (
        (W1a, b1a, g1, be1, W1b, b1b, W1c, b1c, Wr1, br1),
        (W2a, b2a, g2, be2, W2b, b2b, W2c, b2c, Wr2, br2),
        (W3a, b3a, g3, be3, W3b, b3b, W3c, b3c, Wr3, br3))
    for r, (wa, ba, g, be, wb, bb, wc, bc, wr, br) in enumerate(rounds):
        parts = agg_parts(h)
        t, ident, stats = _round_a(h, parts, wa, ba, wr, br)
        if r < 2:
            h = _round_b(t, ident, stats, g, be, wb, bb, wc, bc)
        else:
            h, p4 = _round_b3(t, ident, stats, g, be, wb, bb, wc, bc, W4)
    parts4 = segsum64(p4, src, dst).reshape(2, _N, _DOUT)
    return _final(p4, parts4, b4)


# speedup vs baseline: 2.2844x; 1.0212x over previous
import functools

import jax
import jax.numpy as jnp
from jax import lax
from jax.experimental import pallas as pl
from jax.experimental.pallas import tpu as pltpu
from jax.experimental.pallas import tpu_sc as plsc

_N = 10000
_E = 320000
_H = 128
_DOUT = 64

_NCORES = 2
_NSUB = 16
_NTILES = _NCORES * _NSUB
_EPT = _E // _NTILES
_EC = 80
_NEC = _EPT // _EC
_RC = 80
_NRC = _N // _RC
_RCPS = (_NRC + _NSUB - 1) // _NSUB

_BLK = 2000
_NB = _N // _BLK


def _make_segsum(w=_H):
    def _segsum_body(h_hbm, src_hbm, dst_hbm, out_hbm, src_v, dst_v, rows_v,
                     acc, sem):
        c = lax.axis_index("c")
        s = lax.axis_index("s")
        wid = c * _NSUB + s

        zeros16 = jnp.zeros((16,), jnp.float32)

        def _zrow(i, carry):
            for j in range(w // 16):
                rows_v[i, pl.ds(j * 16, 16)] = zeros16
            return carry

        lax.fori_loop(0, _RC, _zrow, 0)

        def _zchunk(j, carry):
            chunk = j * _NSUB + s

            @pl.when(chunk < _NRC)
            def _():
                pltpu.sync_copy(rows_v, acc.at[pl.ds(chunk * _RC, _RC)])

            return carry

        lax.fori_loop(0, _RCPS, _zchunk, 0)

        pltpu.sync_copy(src_hbm.at[wid], src_v)
        pltpu.sync_copy(dst_hbm.at[wid], dst_v)
        plsc.subcore_barrier()

        def _echunk(j, carry):
            pltpu.async_copy(h_hbm.at[src_v.at[j]], rows_v, sem).wait()
            pltpu.sync_copy(rows_v, acc.at[dst_v.at[j]], add=True)
            return carry

        lax.fori_loop(0, _NEC, _echunk, 0)
        plsc.subcore_barrier()

        def _ochunk(j, carry):
            chunk = j * _NSUB + s

            @pl.when(chunk < _NRC)
            def _():
                r0 = chunk * _RC
                pltpu.sync_copy(acc.at[pl.ds(r0, _RC)], rows_v)
                pltpu.sync_copy(rows_v, out_hbm.at[pl.ds(c * _N + r0, _RC)])

            return carry

        lax.fori_loop(0, _RCPS, _ochunk, 0)

    return functools.partial(
        pl.kernel,
        mesh=plsc.VectorSubcoreMesh(core_axis_name="c", subcore_axis_name="s"),
        compiler_params=pltpu.CompilerParams(use_tc_tiling_on_sc=False),
        out_type=jax.ShapeDtypeStruct((2 * _N, w), jnp.float32),
        scratch_types=[
            pltpu.VMEM((_NEC, _EC), jnp.int32),
            pltpu.VMEM((_NEC, _EC), jnp.int32),
            pltpu.VMEM((_EC, w), jnp.float32),
            pltpu.VMEM_SHARED((_N, w), jnp.float32),
            pltpu.SemaphoreType.DMA,
        ],
    )(_segsum_body)


_make_segsum = functools.cache(_make_segsum)


def _round_a_body(h_ref, p_ref, wa_ref, ba_ref, wr_ref, br_ref,
                  t_ref, id_ref, st_ref):
    j = pl.program_id(0)
    h = h_ref[...]
    agg = h + p_ref[0] + p_ref[1]
    t = jnp.dot(agg, wa_ref[...], preferred_element_type=jnp.float32) + ba_ref[...]
    t_ref[...] = t
    id_ref[...] = jnp.dot(h, wr_ref[...], preferred_element_type=jnp.float32) + br_ref[...]
    blk = jnp.concatenate(
        [jnp.sum(t, axis=0, keepdims=True),
         jnp.sum(t * t, axis=0, keepdims=True)], axis=0)

    @pl.when(j == 0)
    def _():
        st_ref[...] = blk

    @pl.when(j > 0)
    def _():
        st_ref[...] += blk


def _round_a(h, parts, wa, ba, wr, br):
    return pl.pallas_call(
        _round_a_body,
        grid=(_NB,),
        in_specs=[
            pl.BlockSpec((_BLK, _H), lambda j: (j, 0)),
            pl.BlockSpec((2, _BLK, _H), lambda j: (0, j, 0)),
            pl.BlockSpec((_H, _H), lambda j: (0, 0)),
            pl.BlockSpec((1, _H), lambda j: (0, 0)),
            pl.BlockSpec((_H, _H), lambda j: (0, 0)),
            pl.BlockSpec((1, _H), lambda j: (0, 0)),
        ],
        out_specs=[
            pl.BlockSpec((_BLK, _H), lambda j: (j, 0)),
            pl.BlockSpec((_BLK, _H), lambda j: (j, 0)),
            pl.BlockSpec((2, _H), lambda j: (0, 0)),
        ],
        out_shape=[
            jax.ShapeDtypeStruct((_N, _H), jnp.float32),
            jax.ShapeDtypeStruct((_N, _H), jnp.float32),
            jax.ShapeDtypeStruct((2, _H), jnp.float32),
        ],
    )(h, parts, wa, ba.reshape(1, _H), wr, br.reshape(1, _H))


def _round_b_body(t_ref, id_ref, st_ref, g_ref, be_ref, wb_ref, bb_ref,
                  wc_ref, bc_ref, o_ref):
    st = st_ref[...]
    m = st[0:1] * (1.0 / _N)
    v = st[1:2] * (1.0 / _N) - m * m
    inv = lax.rsqrt(v + 1e-5)
    u = (t_ref[...] - m) * (inv * g_ref[...]) + be_ref[...]
    u = jnp.maximum(u, 0.0)
    u = jnp.maximum(
        jnp.dot(u, wb_ref[...], preferred_element_type=jnp.float32) + bb_ref[...], 0.0)
    hh = jnp.dot(u, wc_ref[...], preferred_element_type=jnp.float32) + bc_ref[...]
    o_ref[...] = jnp.maximum(hh + id_ref[...], 0.0)


def _round_b(t, ident, stats, g, be, wb, bb, wc, bc):
    return pl.pallas_call(
        _round_b_body,
        grid=(_NB,),
        in_specs=[
            pl.BlockSpec((_BLK, _H), lambda j: (j, 0)),
            pl.BlockSpec((_BLK, _H), lambda j: (j, 0)),
            pl.BlockSpec((2, _H), lambda j: (0, 0)),
            pl.BlockSpec((1, _H), lambda j: (0, 0)),
            pl.BlockSpec((1, _H), lambda j: (0, 0)),
            pl.BlockSpec((_H, _H), lambda j: (0, 0)),
            pl.BlockSpec((1, _H), lambda j: (0, 0)),
            pl.BlockSpec((_H, _H), lambda j: (0, 0)),
            pl.BlockSpec((1, _H), lambda j: (0, 0)),
        ],
        out_specs=pl.BlockSpec((_BLK, _H), lambda j: (j, 0)),
        out_shape=jax.ShapeDtypeStruct((_N, _H), jnp.float32),
    )(t, ident, stats, g.reshape(1, _H), be.reshape(1, _H),
      wb, bb.reshape(1, _H), wc, bc.reshape(1, _H))


def _round_b3_body(t_ref, id_ref, st_ref, g_ref, be_ref, wb_ref, bb_ref,
                   wc_ref, bc_ref, w4_ref, o_ref, p_ref):
    st = st_ref[...]
    m = st[0:1] * (1.0 / _N)
    v = st[1:2] * (1.0 / _N) - m * m
    inv = lax.rsqrt(v + 1e-5)
    u = (t_ref[...] - m) * (inv * g_ref[...]) + be_ref[...]
    u = jnp.maximum(u, 0.0)
    u = jnp.maximum(
        jnp.dot(u, wb_ref[...], preferred_element_type=jnp.float32) + bb_ref[...], 0.0)
    hh = jnp.dot(u, wc_ref[...], preferred_element_type=jnp.float32) + bc_ref[...]
    h3 = jnp.maximum(hh + id_ref[...], 0.0)
    o_ref[...] = h3
    p_ref[...] = jnp.dot(h3, w4_ref[...], preferred_element_type=jnp.float32)


def _round_b3(t, ident, stats, g, be, wb, bb, wc, bc, w4):
    return pl.pallas_call(
        _round_b3_body,
        grid=(_NB,),
        in_specs=[
            pl.BlockSpec((_BLK, _H), lambda j: (j, 0)),
            pl.BlockSpec((_BLK, _H), lambda j: (j, 0)),
            pl.BlockSpec((2, _H), lambda j: (0, 0)),
            pl.BlockSpec((1, _H), lambda j: (0, 0)),
            pl.BlockSpec((1, _H), lambda j: (0, 0)),
            pl.BlockSpec((_H, _H), lambda j: (0, 0)),
            pl.BlockSpec((1, _H), lambda j: (0, 0)),
            pl.BlockSpec((_H, _H), lambda j: (0, 0)),
            pl.BlockSpec((1, _H), lambda j: (0, 0)),
            pl.BlockSpec((_H, _DOUT), lambda j: (0, 0)),
        ],
        out_specs=[
            pl.BlockSpec((_BLK, _H), lambda j: (j, 0)),
            pl.BlockSpec((_BLK, _DOUT), lambda j: (j, 0)),
        ],
        out_shape=[
            jax.ShapeDtypeStruct((_N, _H), jnp.float32),
            jax.ShapeDtypeStruct((_N, _DOUT), jnp.float32),
        ],
    )(t, ident, stats, g.reshape(1, _H), be.reshape(1, _H),
      wb, bb.reshape(1, _H), wc, bc.reshape(1, _H), w4)


def _final_body(p_ref, parts_ref, b4_ref, o_ref):
    z = p_ref[...] + parts_ref[0] + parts_ref[1] + b4_ref[...]
    mx = jnp.max(z, axis=1, keepdims=True)
    e = jnp.exp(z - mx)
    o_ref[...] = z - mx - jnp.log(jnp.sum(e, axis=1, keepdims=True))


def _final(p4, parts, b4):
    return pl.pallas_call(
        _final_body,
        grid=(_NB,),
        in_specs=[
            pl.BlockSpec((_BLK, _DOUT), lambda j: (j, 0)),
            pl.BlockSpec((2, _BLK, _DOUT), lambda j: (0, j, 0)),
            pl.BlockSpec((1, _DOUT), lambda j: (0, 0)),
        ],
        out_specs=pl.BlockSpec((_BLK, _DOUT), lambda j: (j, 0)),
        out_shape=jax.ShapeDtypeStruct((_N, _DOUT), jnp.float32),
    )(p4, parts, b4.reshape(1, _DOUT))


def kernel(x, edge_index,
           W1a, b1a, g1, be1, W1b, b1b, W1c, b1c,
           W2a, b2a, g2, be2, W2b, b2b, W2c, b2c,
           W3a, b3a, g3, be3, W3b, b3b, W3c, b3c,
           W4, b4, Wr1, br1, Wr2, br2, Wr3, br3):
    src = edge_index[0].reshape(_NTILES, _NEC, _EC)
    dst = edge_index[1].reshape(_NTILES, _NEC, _EC)

    segsum = _make_segsum()
    segsum64 = _make_segsum(_DOUT)

    def agg_parts(h):
        return segsum(h, src, dst).reshape(2, _N, _H)

    h = x
    p4 = None
    rounds = (
        (W1a, b1a, g1, be1, W1b, b1b, W1c, b1c, Wr1, br1),
        (W2a, b2a, g2, be2, W2b, b2b, W2c, b2c, Wr2, br2),
        (W3a, b3a, g3, be3, W3b, b3b, W3c, b3c, Wr3, br3))
    for r, (wa, ba, g, be, wb, bb, wc, bc, wr, br) in enumerate(rounds):
        parts = agg_parts(h)
        t, ident, stats = _round_a(h, parts, wa, ba, wr, br)
        if r < 2:
            h = _round_b(t, ident, stats, g, be, wb, bb, wc, bc)
        else:
            h, p4 = _round_b3(t, ident, stats, g, be, wb, bb, wc, bc, W4)
    parts4 = segsum64(p4, src, dst).reshape(2, _N, _DOUT)
    return _final(p4, parts4, b4)
